# Initial kernel scaffold; baseline (speedup 1.0000x reference)
import jax
import jax.numpy as jnp
from jax.experimental import pallas as pl


def kernel(node_features, edge_index, edge_type, edge_norm, weight):
    raise NotImplementedError("write your pallas kernel here")


# speedup vs baseline: 20.2185x; 20.2185x over previous
import functools

import jax
import jax.numpy as jnp
from jax import lax
from jax.experimental import pallas as pl
from jax.experimental.pallas import tpu as pltpu
from jax.experimental.pallas import tpu_sc as plsc

_N = 10000
_E = 320000
_D = 128
_R = 16

_NC = 2
_NS = 16
_NW = _NC * _NS
_EPW = _E // _NW
_C = 80
_NCHUNK = _EPW // _C
_NPAD = 10240
_RZ = _NPAD // _NS


def _mm_body(h_ref, w_ref, y_ref):
    y_ref[0] = jnp.dot(h_ref[...], w_ref[0], preferred_element_type=jnp.float32)


def _relation_matmul(node_features, weight):
    bn = 1000
    return pl.pallas_call(
        _mm_body,
        grid=(_R, _N // bn),
        in_specs=[
            pl.BlockSpec((bn, _D), lambda r, i: (i, 0)),
            pl.BlockSpec((1, _D, _D), lambda r, i: (r, 0, 0)),
        ],
        out_specs=pl.BlockSpec((1, bn, _D), lambda r, i: (r, i, 0)),
        out_shape=jax.ShapeDtypeStruct((_R, _N, _D), jnp.float32),
    )(node_features, weight)


def _sc_body(pack_hbm, normc_hbm, y_hbm, zeros_hbm, out_hbm, edat_v, norm_v, g_v, d_v, rows_v, acc, sem):
    cid = lax.axis_index("c")
    sid = lax.axis_index("s")
    wid = sid * _NC + cid

    pltpu.sync_copy(zeros_hbm, acc.at[pl.ds(sid * _RZ, _RZ)])
    plsc.subcore_barrier()

    def chunk_body(k, carry):
        blk = wid * _NCHUNK + k
        pltpu.sync_copy(pack_hbm.at[blk], edat_v)
        pltpu.sync_copy(normc_hbm.at[blk], norm_v)
        for i in range(_C // 16):
            sl = pl.ds(i * 16, 16)
            g_v[sl] = edat_v[2, sl] * _N + edat_v[0, sl]
            d_v[sl] = edat_v[1, sl]
        pltpu.async_copy(y_hbm.at[g_v], rows_v, sem).wait()

        def scale_body(e, c2):
            nv = plsc.load_gather(norm_v, [jnp.full((16,), e, jnp.int32)])
            for j in range(_D // 16):
                sl = pl.ds(j * 16, 16)
                rows_v[e, sl] = rows_v[e, sl] * nv
            return c2

        lax.fori_loop(0, _C, scale_body, 0)
        pltpu.sync_copy(rows_v, acc.at[d_v], add=True)
        return carry

    lax.fori_loop(0, _NCHUNK, chunk_body, 0)

    plsc.subcore_barrier()

    last_full = _N // _RZ
    rem = _N - last_full * _RZ

    @pl.when(sid < last_full)
    def _():
        pltpu.sync_copy(
            acc.at[pl.ds(sid * _RZ, _RZ)], out_hbm.at[cid, pl.ds(sid * _RZ, _RZ)]
        )

    @pl.when(sid == last_full)
    def _():
        pltpu.sync_copy(
            acc.at[pl.ds(last_full * _RZ, rem)],
            out_hbm.at[cid, pl.ds(last_full * _RZ, rem)],
        )


@functools.cache
def _sc_scatter():
    return pl.kernel(
        _sc_body,
        out_type=jax.ShapeDtypeStruct((_NC, _N, _D), jnp.float32),
        mesh=plsc.VectorSubcoreMesh(
            core_axis_name="c", subcore_axis_name="s", num_cores=_NC, num_subcores=_NS
        ),
        compiler_params=pltpu.CompilerParams(needs_layout_passes=False),
        scratch_types=[
            pltpu.VMEM((3, _C), jnp.int32),
            pltpu.VMEM((_C,), jnp.float32),
            pltpu.VMEM((_C,), jnp.int32),
            pltpu.VMEM((_C,), jnp.int32),
            pltpu.VMEM((_C, _D), jnp.float32),
            pltpu.VMEM_SHARED((_NPAD, _D), jnp.float32),
            pltpu.SemaphoreType.DMA,
        ],
    )


def _add_body(p_ref, o_ref):
    o_ref[...] = p_ref[0] + p_ref[1]


def _merge_partials(partials):
    ba = 1000
    return pl.pallas_call(
        _add_body,
        grid=(_N // ba,),
        in_specs=[pl.BlockSpec((_NC, ba, _D), lambda i: (0, i, 0))],
        out_specs=pl.BlockSpec((ba, _D), lambda i: (i, 0)),
        out_shape=jax.ShapeDtypeStruct((_N, _D), jnp.float32),
    )(partials)


def kernel(node_features, edge_index, edge_type, edge_norm, weight):
    src = edge_index[0]
    dst = edge_index[1]
    pack = jnp.stack(
        [
            src.reshape(_E // _C, _C),
            dst.reshape(_E // _C, _C),
            edge_type.reshape(_E // _C, _C),
        ],
        axis=1,
    )
    normc = edge_norm.reshape(_E // _C, _C)
    y = _relation_matmul(node_features, weight).reshape(_R * _N, _D)
    zeros = jnp.zeros((_RZ, _D), jnp.float32)
    partials = _sc_scatter()(pack, normc, y, zeros)
    return _merge_partials(partials)
